# baseline (speedup 1.0000x reference)
import functools

import jax
import jax.numpy as jnp
from jax import lax
from jax.experimental import pallas as pl
from jax.experimental.pallas import tpu as pltpu
from jax.experimental.pallas import tpu_sc as plsc

_N_CHANNELS = 192
_N_KNOTS = 63
_T_RANGE = 4.0


def _grid_value():
    round_to = 1e-06
    return float(_T_RANGE) / (_N_KNOTS // 2) // round_to * round_to


def kernel(x, coefficients_vect):
    B, C, H, W = x.shape
    n_pages = B * C
    n_workers = 32
    pages_per_w = n_pages // n_workers

    grid = _grid_value()
    inv_grid = 1.0 / grid
    half = _N_KNOTS // 2
    n_tab = _N_CHANNELS * _N_KNOTS

    g0 = coefficients_vect
    g1 = jnp.concatenate([coefficients_vect[1:], jnp.zeros((1,), jnp.float32)])
    g2 = jnp.concatenate([coefficients_vect[2:], jnp.zeros((2,), jnp.float32)])
    fl = (jnp.arange(n_tab, dtype=jnp.float32) % _N_KNOTS) - float(half)
    a = 0.5 * (g0 + g1)
    b = g1 - g0
    c2 = 0.5 * (g0 + g2) - g1
    p2_t = c2
    p1_t = b - 2.0 * c2 * fl
    p0_t = a - b * fl + c2 * fl * fl

    mesh = plsc.VectorSubcoreMesh(core_axis_name="c", subcore_axis_name="s")

    @functools.partial(
        pl.kernel,
        mesh=mesh,
        out_type=jax.ShapeDtypeStruct((B, C, H, W), jnp.float32),
        scratch_types=[
            pltpu.VMEM((n_tab,), jnp.float32),
            pltpu.VMEM((n_tab,), jnp.float32),
            pltpu.VMEM((n_tab,), jnp.float32),
            pltpu.VMEM((H, W), jnp.float32),
            pltpu.VMEM((H, W), jnp.float32),
            pltpu.VMEM((H, W), jnp.float32),
            pltpu.VMEM((H, W), jnp.float32),
            pltpu.SemaphoreType.DMA,
            pltpu.SemaphoreType.DMA,
            pltpu.SemaphoreType.DMA,
            pltpu.SemaphoreType.DMA,
        ],
        compiler_params=pltpu.CompilerParams(needs_layout_passes=False),
    )
    def spline_sc(x_hbm, p0_hbm, p1_hbm, p2_hbm, out_hbm,
                  tab0, tab1, tab2, xb0, xb1, ob0, ob1,
                  sin0, sin1, sout0, sout1):
        wid = lax.axis_index("s") * 2 + lax.axis_index("c")
        pltpu.sync_copy(p0_hbm, tab0)
        pltpu.sync_copy(p1_hbm, tab1)
        pltpu.sync_copy(p2_hbm, tab2)
        xbufs, obufs = (xb0, xb1), (ob0, ob1)
        sins, souts = (sin0, sin1), (sout0, sout1)
        n_pairs = pages_per_w // 2

        wb = lax.div(wid, 2)
        c_base = lax.rem(wid, 2) * pages_per_w

        pltpu.async_copy(x_hbm.at[wb, c_base], xb0, sin0)
        pltpu.async_copy(x_hbm.at[wb, c_base + 1], xb1, sin1)

        def pair_body(g, carry):
            for bi in range(2):
                pb, pc = wb, c_base + g * 2 + bi
                xbuf, obuf = xbufs[bi], obufs[bi]
                sin, sout = sins[bi], souts[bi]
                pltpu.make_async_copy(x_hbm.at[pb, pc], xbuf, sin).wait()

                @pl.when(g > 0)
                def _wait_out():
                    pltpu.make_async_copy(obuf, out_hbm.at[pb, pc], sout).wait()

                base = pc * _N_KNOTS + half - 64

                @plsc.parallel_loop(0, H, 1, unroll=8)
                def vec_body(r):
                    for cc in range(W // 16):
                        xv = xbuf[r, pl.ds(cc * 16, 16)]
                        q = xv * inv_grid
                        ti = (q + 64.0).astype(jnp.int32)
                        ti = jnp.minimum(jnp.maximum(ti, 64 - half),
                                         64 + half - 2)
                        idx = ti + base
                        p0 = plsc.load_gather(tab0, [idx])
                        p1 = plsc.load_gather(tab1, [idx])
                        p2 = plsc.load_gather(tab2, [idx])
                        obuf[r, pl.ds(cc * 16, 16)] = p0 + q * (p1 + q * p2)

                pltpu.async_copy(obuf, out_hbm.at[pb, pc], sout)

                @pl.when(g < n_pairs - 1)
                def _next_in():
                    pltpu.async_copy(x_hbm.at[pb, pc + 2], xbuf, sin)
            return carry

        lax.fori_loop(0, n_pairs, pair_body, 0)
        lc = c_base + pages_per_w - 2
        pltpu.make_async_copy(ob0, out_hbm.at[wb, lc], sout0).wait()
        pltpu.make_async_copy(ob1, out_hbm.at[wb, lc + 1], sout1).wait()

    return spline_sc(x, p0_t, p1_t, p2_t)


# speedup vs baseline: 1.2692x; 1.2692x over previous
import functools

import jax
import jax.numpy as jnp
from jax import lax
from jax.experimental import pallas as pl
from jax.experimental.pallas import tpu as pltpu
from jax.experimental.pallas import tpu_sc as plsc

_N_CHANNELS = 192
_N_KNOTS = 63
_T_RANGE = 4.0


def _grid_value():
    round_to = 1e-06
    return float(_T_RANGE) / (_N_KNOTS // 2) // round_to * round_to


def kernel(x, coefficients_vect):
    B, C, H, W = x.shape
    n_pages = B * C
    n_workers = 32
    pages_per_w = n_pages // n_workers

    grid = _grid_value()
    inv_grid = 1.0 / grid
    half = _N_KNOTS // 2
    n_tab = _N_CHANNELS * _N_KNOTS

    g0 = coefficients_vect
    g1 = jnp.concatenate([coefficients_vect[1:], jnp.zeros((1,), jnp.float32)])
    g2 = jnp.concatenate([coefficients_vect[2:], jnp.zeros((2,), jnp.float32)])
    fl = (jnp.arange(n_tab, dtype=jnp.float32) % _N_KNOTS) - float(half)
    a = 0.5 * (g0 + g1)
    b = g1 - g0
    c2 = 0.5 * (g0 + g2) - g1
    p2_t = c2
    p1_t = b - 2.0 * c2 * fl
    p0_t = a - b * fl + c2 * fl * fl

    mesh = plsc.VectorSubcoreMesh(core_axis_name="c", subcore_axis_name="s")

    @functools.partial(
        pl.kernel,
        mesh=mesh,
        out_type=jax.ShapeDtypeStruct((B, C, H, W), jnp.float32),
        scratch_types=[
            pltpu.VMEM((n_tab,), jnp.float32),
            pltpu.VMEM((n_tab,), jnp.float32),
            pltpu.VMEM((n_tab,), jnp.float32),
            pltpu.VMEM((2, H, W), jnp.float32),
            pltpu.VMEM((2, H, W), jnp.float32),
            pltpu.VMEM((H, W), jnp.float32),
            pltpu.VMEM((H, W), jnp.float32),
            pltpu.SemaphoreType.DMA,
            pltpu.SemaphoreType.DMA,
            pltpu.SemaphoreType.DMA,
            pltpu.SemaphoreType.DMA,
        ],
        compiler_params=pltpu.CompilerParams(needs_layout_passes=False),
    )
    def spline_sc(x_hbm, p0_hbm, p1_hbm, p2_hbm, out_hbm,
                  tab0, tab1, tab2, xb0, xb1, ob0, ob1,
                  sin0, sin1, sout0, sout1):
        wid = lax.axis_index("s") * 2 + lax.axis_index("c")
        pltpu.sync_copy(p0_hbm, tab0)
        pltpu.sync_copy(p1_hbm, tab1)
        pltpu.sync_copy(p2_hbm, tab2)
        xbufs, obufs = (xb0, xb1), (ob0, ob1)
        sins, souts = (sin0, sin1), (sout0, sout1)
        n_quads = pages_per_w // 4

        wb = lax.div(wid, 2)
        c_base = lax.rem(wid, 2) * pages_per_w

        pltpu.async_copy(x_hbm.at[wb, pl.ds(c_base, 2)], xb0, sin0)
        pltpu.async_copy(x_hbm.at[wb, pl.ds(c_base + 2, 2)], xb1, sin1)

        def quad_body(g, carry):
            for bi in range(2):
                pc = c_base + g * 4 + bi * 2
                xbuf, sin = xbufs[bi], sins[bi]
                pltpu.make_async_copy(
                    x_hbm.at[wb, pl.ds(pc, 2)], xbuf, sin).wait()

                for p in range(2):
                    obuf, sout = obufs[p], souts[p]
                    if bi == 1:
                        pltpu.make_async_copy(
                            obuf, out_hbm.at[wb, pc + p - 2], sout).wait()
                    else:
                        @pl.when(g > 0)
                        def _wait_out(obuf=obuf, sout=sout, pg=pc + p - 2):
                            pltpu.make_async_copy(
                                obuf, out_hbm.at[wb, pg], sout).wait()

                    base = (pc + p) * _N_KNOTS + half - 64

                    @plsc.parallel_loop(0, H, 1, unroll=4)
                    def vec_body(r, p=p, base=base, xbuf=xbuf, obuf=obuf):
                        for cc in range(W // 16):
                            xv = xbuf[p, r, pl.ds(cc * 16, 16)]
                            q = xv * inv_grid
                            ti = (q + 64.0).astype(jnp.int32)
                            ti = jnp.minimum(jnp.maximum(ti, 64 - half),
                                             64 + half - 2)
                            idx = ti + base
                            p0 = plsc.load_gather(tab0, [idx])
                            p1 = plsc.load_gather(tab1, [idx])
                            p2 = plsc.load_gather(tab2, [idx])
                            obuf[r, pl.ds(cc * 16, 16)] = (
                                p0 + q * (p1 + q * p2))

                    pltpu.async_copy(obuf, out_hbm.at[wb, pc + p], sout)

                @pl.when(g < n_quads - 1)
                def _next_in(xbuf=xbuf, sin=sin, pc=pc):
                    pltpu.async_copy(
                        x_hbm.at[wb, pl.ds(pc + 4, 2)], xbuf, sin)
            return carry

        lax.fori_loop(0, n_quads, quad_body, 0)
        lc = c_base + pages_per_w - 2
        pltpu.make_async_copy(ob0, out_hbm.at[wb, lc], sout0).wait()
        pltpu.make_async_copy(ob1, out_hbm.at[wb, lc + 1], sout1).wait()

    return spline_sc(x, p0_t, p1_t, p2_t)


# speedup vs baseline: 1.5239x; 1.2007x over previous
import functools

import jax
import jax.numpy as jnp
from jax import lax
from jax.experimental import pallas as pl
from jax.experimental.pallas import tpu as pltpu
from jax.experimental.pallas import tpu_sc as plsc

_N_CHANNELS = 192
_N_KNOTS = 63
_T_RANGE = 4.0


def _grid_value():
    round_to = 1e-06
    return float(_T_RANGE) / (_N_KNOTS // 2) // round_to * round_to


def kernel(x, coefficients_vect):
    B, C, H, W = x.shape
    n_pages = B * C
    n_workers = 32
    pages_per_w = n_pages // n_workers

    grid = _grid_value()
    inv_grid = 1.0 / grid
    half = _N_KNOTS // 2
    n_tab = _N_CHANNELS * _N_KNOTS

    g0 = coefficients_vect
    g1 = jnp.concatenate([coefficients_vect[1:], jnp.zeros((1,), jnp.float32)])
    g2 = jnp.concatenate([coefficients_vect[2:], jnp.zeros((2,), jnp.float32)])
    fl = (jnp.arange(n_tab, dtype=jnp.float32) % _N_KNOTS) - float(half)
    a = 0.5 * (g0 + g1)
    b = g1 - g0
    c2 = 0.5 * (g0 + g2) - g1
    p2_t = c2
    p1_t = b - 2.0 * c2 * fl
    p0_t = a - b * fl + c2 * fl * fl

    mesh = plsc.VectorSubcoreMesh(core_axis_name="c", subcore_axis_name="s")

    @functools.partial(
        pl.kernel,
        mesh=mesh,
        out_type=jax.ShapeDtypeStruct((B, C, H, W), jnp.float32),
        scratch_types=[
            pltpu.VMEM((n_tab,), jnp.float32),
            pltpu.VMEM((n_tab,), jnp.float32),
            pltpu.VMEM((n_tab,), jnp.float32),
            pltpu.VMEM((H, W), jnp.float32),
            pltpu.VMEM((H, W), jnp.float32),
            pltpu.VMEM((H, W), jnp.float32),
            pltpu.VMEM((H, W), jnp.float32),
            pltpu.SemaphoreType.DMA,
            pltpu.SemaphoreType.DMA,
            pltpu.SemaphoreType.DMA,
            pltpu.SemaphoreType.DMA,
        ],
        compiler_params=pltpu.CompilerParams(needs_layout_passes=False),
    )
    def spline_sc(x_hbm, p0_hbm, p1_hbm, p2_hbm, out_hbm,
                  tab0, tab1, tab2, xb0, xb1, ob0, ob1,
                  sin0, sin1, sout0, sout1):
        wid = lax.axis_index("s") * 2 + lax.axis_index("c")
        pltpu.sync_copy(p0_hbm, tab0)
        pltpu.sync_copy(p1_hbm, tab1)
        pltpu.sync_copy(p2_hbm, tab2)
        xbufs, obufs = (xb0, xb1), (ob0, ob1)
        sins, souts = (sin0, sin1), (sout0, sout1)
        n_pairs = pages_per_w // 2

        wb = lax.div(wid, 2)
        c_base = lax.rem(wid, 2) * pages_per_w

        pltpu.async_copy(x_hbm.at[wb, c_base], xb0, sin0)
        pltpu.async_copy(x_hbm.at[wb, c_base + 1], xb1, sin1)

        def pair_body(g, carry):
            for bi in range(2):
                pb, pc = wb, c_base + g * 2 + bi
                xbuf, obuf = xbufs[bi], obufs[bi]
                sin, sout = sins[bi], souts[bi]
                pltpu.make_async_copy(x_hbm.at[pb, pc], xbuf, sin).wait()

                @pl.when(g > 0)
                def _wait_out():
                    pltpu.make_async_copy(obuf, out_hbm.at[pb, pc], sout).wait()

                base = pc * _N_KNOTS + half - 64

                @plsc.parallel_loop(0, H, 1, unroll=4)
                def vec_body(r):
                    for cc in range(W // 16):
                        xv = xbuf[r, pl.ds(cc * 16, 16)]
                        q = xv * inv_grid
                        ti = (q + 64.0).astype(jnp.int32)
                        ti = jnp.minimum(jnp.maximum(ti, 64 - half),
                                         64 + half - 2)
                        idx = ti + base
                        p0 = plsc.load_gather(tab0, [idx])
                        p1 = plsc.load_gather(tab1, [idx])
                        p2 = plsc.load_gather(tab2, [idx])
                        obuf[r, pl.ds(cc * 16, 16)] = p0 + q * (p1 + q * p2)

                pltpu.async_copy(obuf, out_hbm.at[pb, pc], sout)

                @pl.when(g < n_pairs - 1)
                def _next_in():
                    pltpu.async_copy(x_hbm.at[pb, pc + 2], xbuf, sin)
            return carry

        lax.fori_loop(0, n_pairs, pair_body, 0)
        lc = c_base + pages_per_w - 2
        pltpu.make_async_copy(ob0, out_hbm.at[wb, lc], sout0).wait()
        pltpu.make_async_copy(ob1, out_hbm.at[wb, lc + 1], sout1).wait()

    return spline_sc(x, p0_t, p1_t, p2_t)
